# baseline (speedup 1.0000x reference)
import functools

import jax
import jax.numpy as jnp
import numpy as np
from jax import lax
from jax.experimental import pallas as pl
from jax.experimental.pallas import tpu as pltpu
from jax.experimental.pallas import tpu_sc as plsc

_AVG_LOG = float(np.log(33.0))
_BN_SCALE = 1.0 / np.sqrt(1.0 + 1e-5)
_G = 512
_NUM_T = 100
_N = 10000
_E = 320000
_NW = 32
_R = 320
_NPAD = _NW * _R
_CH = 2000
_GR = 128
_FMAX = 3.4e38


_VARIANT = 0


def _seg_body(FP, *refs):
    (dst_hbm, src_hbm, b_hbm, c_hbm,
     s_hbm, q_hbm, mx_hbm, mn_hbm,
     dstb, srcb, pdl, psrc, peid, brows, crows,
     acc_s, acc_q, acc_mx, acc_mn, sem1, sem2) = refs
    ngrp = FP // 16
    if _VARIANT:
        _seg_body_variant(_VARIANT, FP, refs)
        return
    wid = lax.axis_index("s") * 2 + lax.axis_index("c")
    lo = wid * _R

    zero = jnp.zeros((16,), jnp.float32)
    neg = jnp.full((16,), -_FMAX, jnp.float32)
    pos = jnp.full((16,), _FMAX, jnp.float32)

    def init_row(i, _):
        for fg in range(ngrp):
            sl = pl.ds(fg * 16, 16)
            acc_s[i, sl] = zero
            acc_q[i, sl] = zero
            acc_mx[i, sl] = neg
            acc_mn[i, sl] = pos
        return 0

    lax.fori_loop(0, _R, init_row, 0)

    zi = jnp.zeros((16,), jnp.int32)

    def init_idx(i, _):
        for cg in range(_GR // 16):
            sl = pl.ds(cg * 16, 16)
            psrc[i, sl] = zi
            peid[i, sl] = zi
        return 0

    lax.fori_loop(0, 16, init_idx, 0)

    lane = lax.iota(jnp.int32, 16)

    def chunk_body(kk, _):
        base = kk * _CH
        pltpu.sync_copy(dst_hbm.at[pl.ds(base, _CH)], dstb)
        pltpu.sync_copy(src_hbm.at[pl.ds(base, _CH)], srcb)

        def scan_v(v, npend):
            d = dstb[pl.ds(v * 16, 16)]
            s = srcb[pl.ds(v * 16, 16)]
            m = (d >= lo) & (d < lo + _R)
            mi = m.astype(jnp.int32)
            pos = npend + plsc.cumsum(mi) - 1
            row = lax.shift_right_logical(pos, 7)
            col = lax.bitwise_and(pos, 127)
            plsc.store_scatter(pdl, [pos], d - lo, mask=m)
            plsc.store_scatter(psrc, [row, col], s, mask=m)
            eid = (base + v * 16) + lane
            plsc.store_scatter(peid, [row, col], eid, mask=m)
            return npend + jnp.sum(mi)

        npend = lax.fori_loop(0, _CH // 16, scan_v, 0)
        ngr = (npend + _GR - 1) // _GR

        def group_body(g, _):
            gb = g * _GR
            cp1 = pltpu.async_copy(b_hbm.at[psrc.at[g]], brows, sem1)
            cp2 = pltpu.async_copy(c_hbm.at[peid.at[g]], crows, sem2)
            cp1.wait()
            cp2.wait()
            ne = jnp.minimum(_GR, npend - gb)

            def edge_body(e, _):
                dl = pdl[pl.ds(gb + e, 16)][0]
                for fg in range(ngrp):
                    sl = pl.ds(fg * 16, 16)
                    t = brows[e, sl] + crows[e, sl]
                    plsc.addupdate(acc_s.at[dl, sl], t)
                    plsc.addupdate(acc_q.at[dl, sl], t * t)
                    acc_mx[dl, sl] = jnp.maximum(acc_mx[dl, sl], t)
                    acc_mn[dl, sl] = jnp.minimum(acc_mn[dl, sl], t)
                return 0

            lax.fori_loop(0, ne, edge_body, 0)
            return 0

        lax.fori_loop(0, ngr, group_body, 0)
        return 0

    lax.fori_loop(0, _E // _CH, chunk_body, 0)

    pltpu.sync_copy(acc_s, s_hbm.at[pl.ds(lo, _R)])
    pltpu.sync_copy(acc_q, q_hbm.at[pl.ds(lo, _R)])
    pltpu.sync_copy(acc_mx, mx_hbm.at[pl.ds(lo, _R)])
    pltpu.sync_copy(acc_mn, mn_hbm.at[pl.ds(lo, _R)])


def _seg_body_variant(var, FP, refs):
    (dst_hbm, src_hbm, b_hbm, c_hbm,
     s_hbm, q_hbm, mx_hbm, mn_hbm,
     dstb, srcb, pdl, psrc, peid, brows, crows,
     acc_s, acc_q, acc_mx, acc_mn, sem1, sem2) = refs
    ngrp = FP // 16
    wid = lax.axis_index("s") * 2 + lax.axis_index("c")
    lo = 0 if var in (4, 5, 45, 46) else wid * _R
    lane = lax.iota(jnp.int32, 16)

    def chunk_body(kk, _):
        base = kk * _CH
        pltpu.sync_copy(dst_hbm.at[pl.ds(base, _CH)], dstb)
        pltpu.sync_copy(src_hbm.at[pl.ds(base, _CH)], srcb)

        def scan_v(v, npend):
            d = dstb[pl.ds(v * 16, 16)]
            s = srcb[pl.ds(v * 16, 16)]
            m = (d >= lo) & (d < lo + _R)
            mi = m.astype(jnp.int32)
            if var == 46:
                return npend + jnp.sum(d)
            if var in (41, 45):
                return npend + jnp.sum(mi)
            pos = npend + plsc.cumsum(mi) - 1
            if var == 42:
                return npend + jnp.sum(pos)
            if var != 44:
                plsc.store_scatter(pdl, [pos], d - lo, mask=m)
            if var == 43:
                return npend + jnp.sum(mi)
            row = lax.shift_right_logical(pos, 7)
            col = lax.bitwise_and(pos, 127)
            plsc.store_scatter(psrc, [row, col], s, mask=m)
            eid = (base + v * 16) + lane
            plsc.store_scatter(peid, [row, col], eid, mask=m)
            return npend + jnp.sum(mi)

        npend = 128 if var == 2 else lax.fori_loop(0, _CH // 16, scan_v, 0)
        ngr = (npend + _GR - 1) // _GR

        def group_body(g, _):
            cp1 = pltpu.async_copy(b_hbm.at[psrc.at[g]], brows, sem1)
            cp2 = pltpu.async_copy(c_hbm.at[peid.at[g]], crows, sem2)
            cp1.wait()
            cp2.wait()
            for fg in range(ngrp):
                sl = pl.ds(fg * 16, 16)
                acc_s[0, sl] = acc_s[0, sl] + brows[0, sl] + crows[0, sl]
            return 0

        if var in (2,):
            lax.fori_loop(0, ngr, group_body, 0)
        return 0

    lax.fori_loop(0, _E // _CH, chunk_body, 0)
    pltpu.sync_copy(acc_s, s_hbm.at[pl.ds(lo, _R)])
    pltpu.sync_copy(acc_q, q_hbm.at[pl.ds(lo, _R)])
    pltpu.sync_copy(acc_mx, mx_hbm.at[pl.ds(lo, _R)])
    pltpu.sync_copy(acc_mn, mn_hbm.at[pl.ds(lo, _R)])


@functools.lru_cache(maxsize=None)
def _make_seg_kernel(FP):
    mesh = plsc.VectorSubcoreMesh(core_axis_name="c", subcore_axis_name="s")
    out_type = [jax.ShapeDtypeStruct((_NPAD, FP), jnp.float32)] * 4
    scratch = [
        pltpu.VMEM((_CH,), jnp.int32),
        pltpu.VMEM((_CH,), jnp.int32),
        pltpu.VMEM((_CH + 48,), jnp.int32),
        pltpu.VMEM((16, _GR), jnp.int32),
        pltpu.VMEM((16, _GR), jnp.int32),
        pltpu.VMEM((_GR, FP), jnp.float32),
        pltpu.VMEM((_GR, FP), jnp.float32),
        pltpu.VMEM((_R, FP), jnp.float32),
        pltpu.VMEM((_R, FP), jnp.float32),
        pltpu.VMEM((_R, FP), jnp.float32),
        pltpu.VMEM((_R, FP), jnp.float32),
    ]
    scratch += [pltpu.SemaphoreType.DMA, pltpu.SemaphoreType.DMA]
    return pl.kernel(
        functools.partial(_seg_body, FP),
        out_type=out_type,
        mesh=mesh,
        scratch_types=scratch,
        compiler_params=pltpu.CompilerParams(
            use_tc_tiling_on_sc=False, needs_layout_passes=False),
    )


def _mm_body(h_ref, w_ref, o_ref):
    o_ref[...] = jax.lax.dot(h_ref[...], w_ref[...],
                             preferred_element_type=jnp.float32)


def _mm(h, w):
    n, fin = h.shape
    fout = w.shape[1]
    return pl.pallas_call(
        _mm_body,
        in_specs=[pl.BlockSpec((n, fin), lambda: (0, 0)),
                  pl.BlockSpec((fin, fout), lambda: (0, 0))],
        out_specs=pl.BlockSpec((n, fout), lambda: (0, 0)),
        out_shape=jax.ShapeDtypeStruct((n, fout), jnp.float32),
    )(h, w)


def _edge_c_body(ea_ref, w_ref, b_ref, o_ref):
    o_ref[...] = jax.lax.dot(ea_ref[...], w_ref[...],
                             preferred_element_type=jnp.float32) + b_ref[...]


def _edge_c(ea, cw, cb):
    e = ea.shape[0]
    f = cw.shape[1]
    blk = 8000
    return pl.pallas_call(
        _edge_c_body,
        grid=(e // blk,),
        in_specs=[pl.BlockSpec((blk, 3), lambda i: (i, 0)),
                  pl.BlockSpec((3, f), lambda i: (0, 0)),
                  pl.BlockSpec((1, f), lambda i: (0, 0))],
        out_specs=pl.BlockSpec((blk, f), lambda i: (i, 0)),
        out_shape=jax.ShapeDtypeStruct((e, f), jnp.float32),
    )(ea, cw, cb.reshape(1, f))


def _node_out_body(h_ref, a_ref, s_ref, q_ref, mx_ref, mn_ref, cnt_ref,
                   woh_ref, w1_ref, w2_ref, w3_ref, bo_ref, wl_ref, bl_ref,
                   g_ref, bt_ref, o_ref):
    cnt = cnt_ref[...]
    deg = jnp.maximum(cnt, 1.0)
    has = cnt > 0
    invd = 1.0 / deg
    a = a_ref[...]
    sd = s_ref[...] * invd
    mean = jnp.where(has, a + sd, 0.0)
    var = q_ref[...] * invd - sd * sd
    std = jnp.sqrt(jax.nn.relu(var) + 1e-5)
    mx = jnp.where(has, a + mx_ref[...], 0.0)
    mn = jnp.where(has, a + mn_ref[...], 0.0)
    aggs = jnp.concatenate([mean, mn, mx, std], axis=-1)
    la = jnp.log(deg + 1.0)
    amp = la * (1.0 / _AVG_LOG)
    att = _AVG_LOG / la
    v = jax.lax.dot(h_ref[...], woh_ref[...], preferred_element_type=jnp.float32)
    v += jax.lax.dot(aggs, w1_ref[...], preferred_element_type=jnp.float32)
    v += jax.lax.dot(aggs * amp, w2_ref[...], preferred_element_type=jnp.float32)
    v += jax.lax.dot(aggs * att, w3_ref[...], preferred_element_type=jnp.float32)
    v += bo_ref[...]
    v = jax.lax.dot(v, wl_ref[...], preferred_element_type=jnp.float32) + bl_ref[...]
    o_ref[...] = jax.nn.relu(v * (_BN_SCALE * g_ref[...]) + bt_ref[...])


def _node_out(h, a, s, q, mx, mn, cnt2d, woh, w1, w2, w3, bo, wl, bl, g, bt):
    n, fin = h.shape
    f = a.shape[1]
    fo = woh.shape[1]
    blk = 2000
    cw = lambda r, c: pl.BlockSpec((r, c), lambda i: (0, 0))
    return pl.pallas_call(
        _node_out_body,
        grid=(n // blk,),
        in_specs=[pl.BlockSpec((blk, fin), lambda i: (i, 0)),
                  pl.BlockSpec((blk, f), lambda i: (i, 0)),
                  pl.BlockSpec((blk, f), lambda i: (i, 0)),
                  pl.BlockSpec((blk, f), lambda i: (i, 0)),
                  pl.BlockSpec((blk, f), lambda i: (i, 0)),
                  pl.BlockSpec((blk, f), lambda i: (i, 0)),
                  pl.BlockSpec((blk, 1), lambda i: (i, 0)),
                  cw(fin, fo), cw(4 * f, fo), cw(4 * f, fo), cw(4 * f, fo),
                  cw(1, fo), cw(fo, fo), cw(1, fo), cw(1, fo), cw(1, fo)],
        out_specs=pl.BlockSpec((blk, fo), lambda i: (i, 0)),
        out_shape=jax.ShapeDtypeStruct((n, fo), jnp.float32),
    )(h, a, s, q, mx, mn, cnt2d, woh, w1, w2, w3,
      bo.reshape(1, fo), wl, bl.reshape(1, fo), g.reshape(1, fo),
      bt.reshape(1, fo))


def _pool_head_body(h_ref, seg_ref, wm_ref, bm_ref, sum_ref, cnt_ref, out_ref):
    i = pl.program_id(0)
    nsteps = pl.num_programs(0)
    blk = h_ref.shape[0]

    @pl.when(i == 0)
    def _init():
        sum_ref[...] = jnp.zeros_like(sum_ref)
        cnt_ref[...] = jnp.zeros_like(cnt_ref)

    seg = seg_ref[0, 0, :]
    gids = jax.lax.broadcasted_iota(jnp.int32, (_G, blk), 0)
    oh = (gids == seg[None, :]).astype(jnp.float32)
    sum_ref[...] += jax.lax.dot(oh, h_ref[...], preferred_element_type=jnp.float32)
    cnt_ref[...] += jnp.sum(oh, axis=1, keepdims=True)

    @pl.when(i == nsteps - 1)
    def _final():
        cnt = jnp.maximum(cnt_ref[...], 1.0)
        pooled = sum_ref[...] / cnt
        out = jax.lax.dot(pooled, wm_ref[...], preferred_element_type=jnp.float32) + bm_ref[...]
        out_ref[...] = jnp.where(out > 0, out, 0.1 * out)


def _pool_head(h, batch_seg, Wm, bm):
    n, f = h.shape
    blk = 2000
    grid = (n // blk,)
    _, _, out = pl.pallas_call(
        _pool_head_body,
        grid=grid,
        in_specs=[
            pl.BlockSpec((blk, f), lambda i: (i, 0)),
            pl.BlockSpec((1, 1, blk), lambda i: (i, 0, 0)),
            pl.BlockSpec((f, _NUM_T), lambda i: (0, 0)),
            pl.BlockSpec((_NUM_T,), lambda i: (0,)),
        ],
        out_specs=[
            pl.BlockSpec((_G, f), lambda i: (0, 0)),
            pl.BlockSpec((_G, 1), lambda i: (0, 0)),
            pl.BlockSpec((_G, _NUM_T), lambda i: (0, 0)),
        ],
        out_shape=[
            jax.ShapeDtypeStruct((_G, f), jnp.float32),
            jax.ShapeDtypeStruct((_G, 1), jnp.float32),
            jax.ShapeDtypeStruct((_G, _NUM_T), jnp.float32),
        ],
    )(h, batch_seg.reshape(n // blk, 1, blk), Wm, bm)
    return out


def _pad_rows_16(w, fin):
    fo = w.shape[1]
    return jnp.zeros((4, 16, fo), jnp.float32).at[:, :fin, :].set(
        w.reshape(4, fin, fo))


def _layer(h, dst, src, edge_attr, We, be, Wp, bp, Wo, bo, Wl, bl, g, bt,
           cnt2d):
    n, fin = h.shape
    f = Wp.shape[1]
    fp = max(f, 16)
    with_cnt = cnt2d is None
    Wp_i, Wp_j, Wp_e = Wp[:fin], Wp[fin:2 * fin], Wp[2 * fin:]
    w2 = jnp.zeros((fin, 2 * fp), jnp.float32)
    w2 = w2.at[:, :f].set(Wp_i).at[:, fp:fp + f].set(Wp_j)
    ab = _mm(h, w2)
    a, b = ab[:, :fp], ab[:, fp:]
    cw = jnp.zeros((3, fp), jnp.float32).at[:, :f].set(We @ Wp_e)
    cb = jnp.zeros((fp,), jnp.float32).at[:f].set(be @ Wp_e + bp)
    if with_cnt:
        cb = cb.at[f].set(1.0)
    c = _edge_c(edge_attr, cw, cb)
    nslab = max(1, fp // 64)
    fs = fp // nslab
    seg = _make_seg_kernel(fs)
    parts = []
    for si in range(nslab):
        bs = b[:, si * fs:(si + 1) * fs] if nslab > 1 else b
        cs = c[:, si * fs:(si + 1) * fs] if nslab > 1 else c
        parts.append(seg(dst, src, bs, cs))
    if nslab > 1:
        S, Q, MX, MN = (jnp.concatenate([p[i] for p in parts], axis=1)
                        for i in range(4))
    else:
        S, Q, MX, MN = parts[0]
    if with_cnt:
        cnt2d = S[:_N, f:f + 1]
    woh = Wo[:fin]
    w1 = Wo[fin:fin + 4 * f]
    w2o = Wo[fin + 4 * f:fin + 8 * f]
    w3 = Wo[fin + 8 * f:]
    if f < 16:
        w1, w2o, w3 = (_pad_rows_16(w, f).reshape(64, -1) for w in (w1, w2o, w3))
    h_next = _node_out(h, a[:_N], S[:_N], Q[:_N], MX[:_N], MN[:_N], cnt2d,
                       woh, w1, w2o, w3, bo, Wl, bl, g, bt)
    return h_next, cnt2d


_TRUNC = 0


def kernel(x, edge_index, edge_attr, batch_seg, We1, be1, Wp1, bp1, Wo1, bo1, Wl1, bl1, g1, bt1, We2, be2, Wp2, bp2, Wo2, bo2, Wl2, bl2, g2, bt2, We3, be3, Wp3, bp3, Wo3, bo3, Wl3, bl3, g3, bt3, Wm, bm):
    src, dst = edge_index[0], edge_index[1]
    if _TRUNC == 1:
        seg = _make_seg_kernel(64)
        b = jnp.tile(x[:, :2], (1, 32))
        c = jnp.tile(edge_attr[:, :2], (1, 32))
        return seg(dst, src, b, c)[0]
    if _TRUNC == 2:
        h, cnt2d = _layer(x, dst, src, edge_attr, We1, be1, Wp1, bp1, Wo1,
                          bo1, Wl1, bl1, g1, bt1, None)
        return h
    h, cnt2d = _layer(x, dst, src, edge_attr, We1, be1, Wp1, bp1, Wo1, bo1,
                      Wl1, bl1, g1, bt1, None)
    h, _ = _layer(h, dst, src, edge_attr, We2, be2, Wp2, bp2, Wo2, bo2,
                  Wl2, bl2, g2, bt2, cnt2d)
    h, _ = _layer(h, dst, src, edge_attr, We3, be3, Wp3, bp3, Wo3, bo3,
                  Wl3, bl3, g3, bt3, cnt2d)
    return _pool_head(h, batch_seg, Wm, bm)
